# Initial kernel scaffold; baseline (speedup 1.0000x reference)
import functools

import jax
import jax.numpy as jnp
from jax import lax
from jax.experimental import pallas as pl
from jax.experimental.pallas import tpu as pltpu
from jax.experimental.pallas import tpu_sc as plsc

_BANK = 1000001
_DIM = 64
_BATCH = 16384
_MOM = 0.9

_NC, _NS = 2, 16
_NW = _NC * _NS
_BPW = _BATCH // _NW
_CH = 128
_NCH = _BPW // _CH

_COPY_ROWS = 16384


def _copy_body(m_ref, o_ref):
    o_ref[...] = m_ref[...]


def _bank_copy(memory):
    grid = (pl.cdiv(_BANK, _COPY_ROWS),)
    return pl.pallas_call(
        _copy_body,
        grid=grid,
        in_specs=[pl.BlockSpec((_COPY_ROWS, _DIM), lambda i: (i, 0))],
        out_specs=pl.BlockSpec((_COPY_ROWS, _DIM), lambda i: (i, 0)),
        out_shape=jax.ShapeDtypeStruct((_BANK, _DIM), jnp.float32),
    )(memory)


@functools.partial(
    pl.kernel,
    out_type=jax.ShapeDtypeStruct((_BATCH, _DIM), jnp.float32),
    mesh=plsc.VectorSubcoreMesh(core_axis_name="c", subcore_axis_name="s"),
    scratch_types=[
        pltpu.VMEM((_NCH, _CH), jnp.int32),
        pltpu.VMEM((_BPW, _DIM), jnp.float32),
        pltpu.VMEM((_BPW, _DIM), jnp.float32),
        pltpu.SemaphoreType.DMA,
    ],
)
def _sc_update(idx_hbm, data_hbm, mem_hbm, upd_ref, avgs_hbm,
               idx_v, rows_v, data_v, sem):
    wid = lax.axis_index("s") * _NC + lax.axis_index("c")
    base = wid * _BPW

    pltpu.sync_copy(idx_hbm.at[pl.ds(wid * _NCH, _NCH)], idx_v)

    for j in range(_NCH):
        pltpu.async_copy(mem_hbm.at[idx_v.at[j]],
                         rows_v.at[pl.ds(j * _CH, _CH)], sem)
    for j in range(_NCH):
        pltpu.make_async_copy(mem_hbm.at[idx_v.at[j]],
                              rows_v.at[pl.ds(j * _CH, _CH)], sem).wait()

    pltpu.sync_copy(rows_v, avgs_hbm.at[pl.ds(base, _BPW)])
    pltpu.sync_copy(data_hbm.at[pl.ds(base, _BPW)], data_v)

    def body(i, carry):
        for k in range(_DIM // 16):
            sl = pl.ds(k * 16, 16)
            rows_v[i, sl] = rows_v[i, sl] * _MOM + data_v[i, sl] * (1.0 - _MOM)
        return carry

    lax.fori_loop(0, _BPW, body, 0)

    for j in range(_NCH):
        pltpu.async_copy(rows_v.at[pl.ds(j * _CH, _CH)],
                         upd_ref.at[idx_v.at[j]], sem)
    for j in range(_NCH):
        pltpu.make_async_copy(rows_v.at[pl.ds(j * _CH, _CH)],
                              upd_ref.at[idx_v.at[j]], sem).wait()


def kernel(idx, data, memory):
    idx2d = idx.astype(jnp.int32).reshape(_NW * _NCH, _CH)
    bank = _bank_copy(memory)
    bank_ref = jax.new_ref(bank)
    avgs = _sc_update(idx2d, data, memory, bank_ref)
    return avgs, bank_ref[...]


# speedup vs baseline: 1.0692x; 1.0692x over previous
import functools

import jax
import jax.numpy as jnp
from jax import lax
from jax.experimental import pallas as pl
from jax.experimental.pallas import tpu as pltpu
from jax.experimental.pallas import tpu_sc as plsc

_BANK = 1000001
_DIM = 64
_BATCH = 16384
_MOM = 0.9

_NC, _NS = 2, 16
_NW = _NC * _NS
_BPW = _BATCH // _NW
_CH = 128
_NCH = _BPW // _CH

_COPY_ROWS = 16384


def _copy_body(m_ref, o_ref):
    o_ref[...] = m_ref[...]


def _bank_copy(memory):
    grid = (pl.cdiv(_BANK, _COPY_ROWS),)
    return pl.pallas_call(
        _copy_body,
        grid=grid,
        in_specs=[pl.BlockSpec((_COPY_ROWS, _DIM), lambda i: (i, 0))],
        out_specs=pl.BlockSpec((_COPY_ROWS, _DIM), lambda i: (i, 0)),
        out_shape=jax.ShapeDtypeStruct((_BANK, _DIM), jnp.float32),
    )(memory)


@functools.partial(
    pl.kernel,
    out_type=jax.ShapeDtypeStruct((_BATCH, _DIM), jnp.float32),
    mesh=plsc.VectorSubcoreMesh(core_axis_name="c", subcore_axis_name="s"),
    compiler_params=pltpu.CompilerParams(use_tc_tiling_on_sc=False),
    scratch_types=[
        pltpu.VMEM((_NCH, _CH), jnp.int32),
        pltpu.VMEM((_BPW, _DIM), jnp.float32),
        pltpu.VMEM((_BPW, _DIM), jnp.float32),
        pltpu.SemaphoreType.DMA,
    ],
)
def _sc_update(idx_hbm, data_hbm, mem_hbm, upd_ref, avgs_hbm,
               idx_v, rows_v, data_v, sem):
    wid = lax.axis_index("s") * _NC + lax.axis_index("c")
    base = wid * _BPW

    pltpu.sync_copy(idx_hbm.at[pl.ds(wid * _NCH, _NCH)], idx_v)

    for j in range(_NCH):
        pltpu.async_copy(mem_hbm.at[idx_v.at[j]],
                         rows_v.at[pl.ds(j * _CH, _CH)], sem)
    for j in range(_NCH):
        pltpu.make_async_copy(mem_hbm.at[idx_v.at[j]],
                              rows_v.at[pl.ds(j * _CH, _CH)], sem).wait()

    pltpu.sync_copy(rows_v, avgs_hbm.at[pl.ds(base, _BPW)])
    pltpu.sync_copy(data_hbm.at[pl.ds(base, _BPW)], data_v)

    def body(i, carry):
        for k in range(_DIM // 16):
            sl = pl.ds(k * 16, 16)
            rows_v[i, sl] = rows_v[i, sl] * _MOM + data_v[i, sl] * (1.0 - _MOM)
        return carry

    lax.fori_loop(0, _BPW, body, 0)

    for j in range(_NCH):
        pltpu.async_copy(rows_v.at[pl.ds(j * _CH, _CH)],
                         upd_ref.at[idx_v.at[j]], sem)
    for j in range(_NCH):
        pltpu.make_async_copy(rows_v.at[pl.ds(j * _CH, _CH)],
                              upd_ref.at[idx_v.at[j]], sem).wait()


def kernel(idx, data, memory):
    idx2d = idx.astype(jnp.int32).reshape(_NW * _NCH, _CH)
    bank = _bank_copy(memory)
    bank_ref = jax.new_ref(bank)
    avgs = _sc_update(idx2d, data, memory, bank_ref)
    return avgs, bank_ref[...]
